# Initial kernel scaffold; baseline (speedup 1.0000x reference)
import jax
import jax.numpy as jnp
from jax.experimental import pallas as pl


def kernel(x, edge_index, edge_weight, W1, W2):
    raise NotImplementedError("write your pallas kernel here")


# speedup vs baseline: 4.9888x; 4.9888x over previous
import functools

import jax
import jax.numpy as jnp
from jax import lax
from jax.experimental import pallas as pl
from jax.experimental.pallas import tpu as pltpu
from jax.experimental.pallas import tpu_sc as plsc

N = 10000
E = 320000
D = 128
H = 128
C = 64

L = 16
NC = 2
NS = 16
NW = NC * NS
CH = 128
NCHUNK = E // CH
KMAX = (NCHUNK + NW - 1) // NW
NP = 10240
RPT = NP // NS
ZR = 128

_mesh = plsc.VectorSubcoreMesh(core_axis_name="c", subcore_axis_name="s")


def _make_spmm(Dm):
    @functools.partial(
        pl.kernel,
        out_type=(jax.ShapeDtypeStruct((NP, Dm), jnp.float32),
                  jax.ShapeDtypeStruct((NP, Dm), jnp.float32)),
        mesh=_mesh,
        scratch_types=[
            pltpu.VMEM((CH,), jnp.int32),
            pltpu.VMEM((CH,), jnp.int32),
            pltpu.VMEM((CH,), jnp.float32),
            pltpu.VMEM((CH, Dm), jnp.float32),
            pltpu.VMEM((ZR, Dm), jnp.float32),
            pltpu.VMEM_SHARED((NP, Dm), jnp.float32),
            pltpu.SemaphoreType.DMA,
        ],
    )
    def spmm(m_hbm, col_hbm, row_hbm, w_hbm, out0_hbm, out1_hbm,
             col_v, row_v, w_v, rows_v, zbuf, acc, sem):
        c = lax.axis_index("c")
        s = lax.axis_index("s")
        wid = c * NS + s

        zvec = jnp.zeros((L,), jnp.float32)

        def zrow(i, carry):
            for j in range(Dm // L):
                zbuf[i, pl.ds(j * L, L)] = zvec
            return carry

        lax.fori_loop(0, ZR, zrow, 0)
        for k in range(RPT // ZR):
            pltpu.sync_copy(zbuf, acc.at[pl.ds(s * RPT + k * ZR, ZR)])
        plsc.subcore_barrier()

        def chunk_body(k, carry):
            cid = wid + k * NW

            @pl.when(cid < NCHUNK)
            def _():
                base = cid * CH
                pltpu.sync_copy(col_hbm.at[pl.ds(base, CH)], col_v)
                pltpu.sync_copy(row_hbm.at[pl.ds(base, CH)], row_v)
                pltpu.sync_copy(w_hbm.at[pl.ds(base, CH)], w_v)
                pltpu.async_copy(m_hbm.at[col_v], rows_v, sem).wait()

                def edge_group(g, icarry):
                    wvec = w_v[pl.ds(g * L, L)]
                    for t in range(L):
                        wgt = wvec[t]
                        i = g * L + t
                        for j in range(Dm // L):
                            rows_v[i, pl.ds(j * L, L)] = (
                                rows_v[i, pl.ds(j * L, L)] * wgt)
                    return icarry

                lax.fori_loop(0, CH // L, edge_group, 0)
                pltpu.sync_copy(rows_v, acc.at[row_v], add=True)

            return carry

        lax.fori_loop(0, KMAX, chunk_body, 0)
        plsc.subcore_barrier()

        @pl.when(c == 0)
        def _():
            for k in range(RPT // ZR):
                r0 = s * RPT + k * ZR
                pltpu.sync_copy(acc.at[pl.ds(r0, ZR)], out0_hbm.at[pl.ds(r0, ZR)])

        @pl.when(c == 1)
        def _():
            for k in range(RPT // ZR):
                r0 = s * RPT + k * ZR
                pltpu.sync_copy(acc.at[pl.ds(r0, ZR)], out1_hbm.at[pl.ds(r0, ZR)])

    return spmm


_spmm128 = _make_spmm(D)

_MM_BLK = 2000


def _mm1(x, w1):
    def body(x_ref, w_ref, o_ref):
        o_ref[...] = jnp.dot(x_ref[...], w_ref[...],
                             preferred_element_type=jnp.float32)

    return pl.pallas_call(
        body,
        grid=(N // _MM_BLK,),
        in_specs=[pl.BlockSpec((_MM_BLK, D), lambda i: (i, 0)),
                  pl.BlockSpec((D, H), lambda i: (0, 0))],
        out_specs=pl.BlockSpec((_MM_BLK, H), lambda i: (i, 0)),
        out_shape=jax.ShapeDtypeStruct((N, H), jnp.float32),
    )(x, w1)


def _relu_add_mm2(s0, s1, w2p):
    def body(a_ref, b_ref, w_ref, o_ref):
        h = jnp.maximum(a_ref[...] + b_ref[...], 0.0)
        o_ref[...] = jnp.dot(h, w_ref[...], preferred_element_type=jnp.float32)

    return pl.pallas_call(
        body,
        grid=(N // _MM_BLK,),
        in_specs=[pl.BlockSpec((_MM_BLK, H), lambda i: (i, 0)),
                  pl.BlockSpec((_MM_BLK, H), lambda i: (i, 0)),
                  pl.BlockSpec((H, D), lambda i: (0, 0))],
        out_specs=pl.BlockSpec((_MM_BLK, D), lambda i: (i, 0)),
        out_shape=jax.ShapeDtypeStruct((N, D), jnp.float32),
    )(s0, s1, w2p)


def _add_partials(t0, t1):
    def body(a_ref, b_ref, o_ref):
        o_ref[...] = a_ref[:, :C] + b_ref[:, :C]

    return pl.pallas_call(
        body,
        grid=(N // _MM_BLK,),
        in_specs=[pl.BlockSpec((_MM_BLK, D), lambda i: (i, 0)),
                  pl.BlockSpec((_MM_BLK, D), lambda i: (i, 0))],
        out_specs=pl.BlockSpec((_MM_BLK, C), lambda i: (i, 0)),
        out_shape=jax.ShapeDtypeStruct((N, C), jnp.float32),
    )(t0, t1)


@jax.jit
def _run(x, col, row, w, w1, w2p):
    p = _mm1(x, w1)
    s0, s1 = _spmm128(p, col, row, w)
    q = _relu_add_mm2(s0, s1, w2p)
    t0, t1 = _spmm128(q, col, row, w)
    return _add_partials(t0, t1)


def kernel(x, edge_index, edge_weight, W1, W2):
    w2p = jnp.pad(W2, ((0, 0), (0, D - C)))
    return _run(x, edge_index[1], edge_index[0], edge_weight, W1, w2p)
